# baseline (speedup 1.0000x reference)
import functools

import jax
import jax.numpy as jnp
from jax import lax
from jax.experimental import pallas as pl
from jax.experimental.pallas import tpu as pltpu
from jax.experimental.pallas import tpu_sc as plsc

B = 32
N_SNPS = 50000
N_GENES = 10000
N_NODES = 160000
N_FILTERS = 8

R_BLK = 2000
N_SNPS_PAD = 52000

CHUNK = 128
TOT_CHUNKS = 1280
NODES_PAD = TOT_CHUNKS * CHUNK
K_GRP = 8
C0 = TOT_CHUNKS // 16
G0 = C0 // K_GRP
N_GENES_PAD = 10240
GENES_PER_TILE = N_GENES_PAD // 16

KB = 1000
N_KSTEPS = N_GENES // KB
H1, H2 = 1024, 256
EPS = 1e-5


def _prep_body(snp_ref, filt_ref, out_ref):
    fmean = jnp.mean(filt_ref[...], axis=0, keepdims=True)
    out_ref[pl.ds(0, N_SNPS), :] = jnp.transpose(snp_ref[...] * fmean)
    out_ref[pl.ds(N_SNPS, N_SNPS_PAD - N_SNPS), :] = jnp.zeros(
        (N_SNPS_PAD - N_SNPS, B), jnp.float32)


def _make_prep():
    return pl.pallas_call(
        _prep_body,
        out_shape=jax.ShapeDtypeStruct((N_SNPS_PAD, B), jnp.float32),
    )


ZROWS = 80


def _sc_body(table_hbm, ids_hbm, genes_hbm, out_hbm,
             ids_v, genes_v, rows_v, zbuf, acc_sh, gsems, ssems, isems):
    c = lax.axis_index("c")
    s = lax.axis_index("s")
    is0 = c == 0

    @pl.when(is0)
    def _core0_work():
        _sc_core0(table_hbm, ids_hbm, genes_hbm, out_hbm,
                  ids_v, genes_v, rows_v, zbuf, acc_sh, gsems, ssems, isems, s)


def _sc_core0(table_hbm, ids_hbm, genes_hbm, out_hbm,
              ids_v, genes_v, rows_v, zbuf, acc_sh, gsems, ssems, isems, s):
    pltpu.async_copy(ids_hbm.at[pl.ds(s * C0, C0)], ids_v, isems.at[0])
    pltpu.async_copy(genes_hbm.at[pl.ds(s * C0, C0)], genes_v, isems.at[1])
    pltpu.make_async_copy(ids_hbm.at[pl.ds(s * C0, C0)], ids_v,
                          isems.at[0]).wait()
    pltpu.make_async_copy(genes_hbm.at[pl.ds(s * C0, C0)], genes_v,
                          isems.at[1]).wait()

    def g_desc(t, p, b):
        return pltpu.make_async_copy(
            table_hbm.at[ids_v.at[t * K_GRP + b]],
            rows_v.at[p * K_GRP + b], gsems.at[p])

    def s_start(t, p, b):
        pltpu.async_copy(rows_v.at[p * K_GRP + b],
                         acc_sh.at[genes_v.at[t * K_GRP + b]],
                         ssems.at[p], add=True)

    def s_wait(t, p, b):
        pltpu.make_async_copy(rows_v.at[p * K_GRP + b],
                              acc_sh.at[genes_v.at[t * K_GRP + b]],
                              ssems.at[p]).wait()

    for b in range(K_GRP):
        g_desc(0, 0, b).start()

    zvec = jnp.zeros((16,), jnp.float32)

    def memset_row(i, carry):
        zbuf[i, pl.ds(0, 16)] = zvec
        zbuf[i, pl.ds(16, 16)] = zvec
        return carry

    lax.fori_loop(0, ZROWS, memset_row, 0)
    z0 = s * GENES_PER_TILE
    for q in range(GENES_PER_TILE // ZROWS):
        pltpu.sync_copy(zbuf, acc_sh.at[pl.ds(z0 + q * ZROWS, ZROWS)])
    plsc.subcore_barrier()

    def group_step(t, carry):
        p = t % 2
        for b in range(K_GRP):
            g_desc(t, p, b).wait()
        for b in range(K_GRP):
            s_start(t, p, b)

        @pl.when(t >= 1)
        def _drain_prev():
            for b in range(K_GRP):
                s_wait(t - 1, 1 - p, b)

        @pl.when(t + 1 < G0)
        def _fire_next():
            for b in range(K_GRP):
                g_desc(t + 1, 1 - p, b).start()

        return carry

    lax.fori_loop(0, G0, group_step, 0)
    for b in range(K_GRP):
        s_wait(G0 - 1, (G0 - 1) % 2, b)
    plsc.subcore_barrier()

    pltpu.sync_copy(acc_sh.at[pl.ds(z0, GENES_PER_TILE)],
                    out_hbm.at[pl.ds(z0, GENES_PER_TILE)])


def _make_sc():
    mesh = plsc.VectorSubcoreMesh(core_axis_name="c", subcore_axis_name="s")
    return functools.partial(
        pl.kernel,
        out_type=jax.ShapeDtypeStruct((N_GENES_PAD, B), jnp.float32),
        mesh=mesh,
        compiler_params=pltpu.CompilerParams(use_tc_tiling_on_sc=False),
        scratch_types=[
            pltpu.VMEM((C0, CHUNK), jnp.int32),
            pltpu.VMEM((C0, CHUNK), jnp.int32),
            pltpu.VMEM((2 * K_GRP, CHUNK, B), jnp.float32),
            pltpu.VMEM((ZROWS, B), jnp.float32),
            pltpu.VMEM_SHARED((N_GENES_PAD, B), jnp.float32),
            pltpu.SemaphoreType.DMA((2,)),
            pltpu.SemaphoreType.DMA((2,)),
            pltpu.SemaphoreType.DMA((3,)),
        ],
    )(_sc_body)


def _mlp_body(parts_ref, w1_ref, b1_ref, g1_ref, be1_ref,
              w2_ref, b2_ref, g2_ref, be2_ref, w3r_ref, b3_ref,
              out_ref, acc_ref):
    k = pl.program_id(0)

    @pl.when(k == 0)
    def _init():
        acc_ref[...] = jnp.zeros_like(acc_ref)

    acc_ref[...] += lax.dot_general(
        parts_ref[...], w1_ref[...], (((0,), (0,)), ((), ())),
        preferred_element_type=jnp.float32)

    @pl.when(k == N_KSTEPS - 1)
    def _finish():
        h1 = acc_ref[...] + b1_ref[...]
        m1 = jnp.mean(h1, axis=0, keepdims=True)
        v1 = jnp.mean((h1 - m1) ** 2, axis=0, keepdims=True)
        h1 = g1_ref[...] * (h1 - m1) * lax.rsqrt(v1 + EPS) + be1_ref[...]
        h1 = jnp.maximum(h1, 0.0)
        h2 = jnp.dot(h1, w2_ref[...], preferred_element_type=jnp.float32) + b2_ref[...]
        m2 = jnp.mean(h2, axis=0, keepdims=True)
        v2 = jnp.mean((h2 - m2) ** 2, axis=0, keepdims=True)
        h2 = g2_ref[...] * (h2 - m2) * lax.rsqrt(v2 + EPS) + be2_ref[...]
        h2 = jnp.maximum(h2, 0.0)
        p = jnp.sum(h2 * w3r_ref[...], axis=1, keepdims=True)
        out_ref[...] = p + b3_ref[...]


def _make_mlp():
    full = lambda i: (0, 0)
    return pl.pallas_call(
        _mlp_body,
        grid=(N_KSTEPS,),
        in_specs=[
            pl.BlockSpec((KB, B), lambda i: (i, 0)),
            pl.BlockSpec((KB, H1), lambda i: (i, 0)),
            pl.BlockSpec((1, H1), full),
            pl.BlockSpec((1, H1), full),
            pl.BlockSpec((1, H1), full),
            pl.BlockSpec((H1, H2), full),
            pl.BlockSpec((1, H2), full),
            pl.BlockSpec((1, H2), full),
            pl.BlockSpec((1, H2), full),
            pl.BlockSpec((1, H2), full),
            pl.BlockSpec((1, 128), full),
        ],
        out_specs=pl.BlockSpec((B, 128), full),
        out_shape=jax.ShapeDtypeStruct((B, 128), jnp.float32),
        scratch_shapes=[pltpu.VMEM((B, H1), jnp.float32)],
    )


def kernel(snp, snp_ids, node_graph_ids, filters, W1, b1, gamma1, beta1,
           W2, b2, gamma2, beta2, W3, b3):
    f32 = jnp.float32

    table = _make_prep()(snp, filters)

    pad = NODES_PAD - N_NODES
    ids_p = jnp.concatenate(
        [snp_ids, jnp.full((pad,), N_SNPS, jnp.int32)]).reshape(
            TOT_CHUNKS, CHUNK)
    genes_p = jnp.concatenate(
        [node_graph_ids, jnp.full((pad,), N_GENES - 1, jnp.int32)]).reshape(
            TOT_CHUNKS, CHUNK)
    parts = _make_sc()(table, ids_p, genes_p)

    out = _make_mlp()(
        parts, W1,
        b1.reshape(1, H1), gamma1.reshape(1, H1), beta1.reshape(1, H1),
        W2, b2.reshape(1, H2), gamma2.reshape(1, H2), beta2.reshape(1, H2),
        W3.reshape(1, H2), jnp.broadcast_to(b3.reshape(1, 1), (1, 128)),
    )
    preds = out[:, :1]
    return (preds, filters)


# speedup vs baseline: 1.0357x; 1.0357x over previous
import functools

import jax
import jax.numpy as jnp
from jax import lax
from jax.experimental import pallas as pl
from jax.experimental.pallas import tpu as pltpu
from jax.experimental.pallas import tpu_sc as plsc

B = 32
N_SNPS = 50000
N_GENES = 10000
N_NODES = 160000
N_FILTERS = 8

R_BLK = 2000
N_SNPS_PAD = 52000

CHUNK = 128
TOT_CHUNKS = 1280
NODES_PAD = TOT_CHUNKS * CHUNK
K_GRP = 8
C0 = 56
C1 = 24
G0 = C0 // K_GRP
G1 = C1 // K_GRP
N_GENES_PAD = 10240
GENES_PER_TILE = N_GENES_PAD // 16

KB = 1000
N_KSTEPS = N_GENES // KB
H1, H2 = 1024, 256
EPS = 1e-5


def _prep_body(snp_ref, filt_ref, out_ref):
    fmean = jnp.mean(filt_ref[...], axis=0, keepdims=True)
    scaled = snp_ref[...] * fmean
    eye = jnp.float32(
        lax.broadcasted_iota(jnp.int32, (B, B), 0)
        == lax.broadcasted_iota(jnp.int32, (B, B), 1))
    t = lax.dot_general(scaled, eye, (((0,), (0,)), ((), ())),
                        preferred_element_type=jnp.float32)
    out_ref[pl.ds(0, N_SNPS), :] = t
    out_ref[pl.ds(N_SNPS, N_SNPS_PAD - N_SNPS), :] = jnp.zeros(
        (N_SNPS_PAD - N_SNPS, B), jnp.float32)


def _make_prep():
    return pl.pallas_call(
        _prep_body,
        out_shape=jax.ShapeDtypeStruct((N_SNPS_PAD, B), jnp.float32),
    )


ZROWS = 80


def _sc_body(table_hbm, ids_hbm, genes_hbm, out_hbm,
             ids_v, genes_v, rows_v, zbuf, acc_sh, gsems, ssems, isems):
    c = lax.axis_index("c")
    s = lax.axis_index("s")
    is0 = c == 0

    @pl.when(is0)
    def _core0_work():
        _sc_worker(table_hbm, ids_hbm, genes_hbm, out_hbm,
                   ids_v, genes_v, rows_v, zbuf, acc_sh, gsems, ssems, isems,
                   s, s * C0, 0, C0, G0)

    @pl.when(~is0)
    def _core1_work():
        _sc_worker(table_hbm, ids_hbm, genes_hbm, out_hbm,
                   ids_v, genes_v, rows_v, zbuf, acc_sh, gsems, ssems, isems,
                   s, 16 * C0 + s * C1, N_GENES_PAD, C1, G1)


def _sc_worker(table_hbm, ids_hbm, genes_hbm, out_hbm,
               ids_v, genes_v, rows_v, zbuf, acc_sh, gsems, ssems, isems,
               s, chunk0, out_base, C, G):
    pltpu.async_copy(ids_hbm.at[pl.ds(chunk0, C)],
                     ids_v.at[pl.ds(0, C)], isems.at[0])
    pltpu.async_copy(genes_hbm.at[pl.ds(chunk0, C)],
                     genes_v.at[pl.ds(0, C)], isems.at[1])
    pltpu.make_async_copy(ids_hbm.at[pl.ds(chunk0, C)],
                          ids_v.at[pl.ds(0, C)], isems.at[0]).wait()
    pltpu.make_async_copy(genes_hbm.at[pl.ds(chunk0, C)],
                          genes_v.at[pl.ds(0, C)], isems.at[1]).wait()

    def g_desc(t, p, b):
        return pltpu.make_async_copy(
            table_hbm.at[ids_v.at[t * K_GRP + b]],
            rows_v.at[p * K_GRP + b], gsems.at[p])

    def s_start(t, p, b):
        pltpu.async_copy(rows_v.at[p * K_GRP + b],
                         acc_sh.at[genes_v.at[t * K_GRP + b]],
                         ssems.at[p], add=True)

    def s_wait(t, p, b):
        pltpu.make_async_copy(rows_v.at[p * K_GRP + b],
                              acc_sh.at[genes_v.at[t * K_GRP + b]],
                              ssems.at[p]).wait()

    for b in range(K_GRP):
        g_desc(0, 0, b).start()

    zvec = jnp.zeros((16,), jnp.float32)

    def memset_row(i, carry):
        zbuf[i, pl.ds(0, 16)] = zvec
        zbuf[i, pl.ds(16, 16)] = zvec
        return carry

    lax.fori_loop(0, ZROWS, memset_row, 0)
    z0 = s * GENES_PER_TILE
    for q in range(GENES_PER_TILE // ZROWS):
        pltpu.sync_copy(zbuf, acc_sh.at[pl.ds(z0 + q * ZROWS, ZROWS)])
    plsc.subcore_barrier()

    def group_step(t, carry):
        p = t % 2
        for b in range(K_GRP):
            g_desc(t, p, b).wait()
        for b in range(K_GRP):
            s_start(t, p, b)

        @pl.when(t >= 1)
        def _drain_prev():
            for b in range(K_GRP):
                s_wait(t - 1, 1 - p, b)

        @pl.when(t + 1 < G)
        def _fire_next():
            for b in range(K_GRP):
                g_desc(t + 1, 1 - p, b).start()

        return carry

    lax.fori_loop(0, G, group_step, 0)
    for b in range(K_GRP):
        s_wait(G - 1, (G - 1) % 2, b)
    plsc.subcore_barrier()

    pltpu.sync_copy(acc_sh.at[pl.ds(z0, GENES_PER_TILE)],
                    out_hbm.at[pl.ds(out_base + z0, GENES_PER_TILE)])


def _make_sc():
    mesh = plsc.VectorSubcoreMesh(core_axis_name="c", subcore_axis_name="s")
    return functools.partial(
        pl.kernel,
        out_type=jax.ShapeDtypeStruct((2 * N_GENES_PAD, B), jnp.float32),
        mesh=mesh,
        compiler_params=pltpu.CompilerParams(use_tc_tiling_on_sc=False),
        scratch_types=[
            pltpu.VMEM((C0, CHUNK), jnp.int32),
            pltpu.VMEM((C0, CHUNK), jnp.int32),
            pltpu.VMEM((2 * K_GRP, CHUNK, B), jnp.float32),
            pltpu.VMEM((ZROWS, B), jnp.float32),
            pltpu.VMEM_SHARED((N_GENES_PAD, B), jnp.float32),
            pltpu.SemaphoreType.DMA((2,)),
            pltpu.SemaphoreType.DMA((2,)),
            pltpu.SemaphoreType.DMA((3,)),
        ],
    )(_sc_body)


def _mlp_body(parts_ref, w1_ref, b1_ref, g1_ref, be1_ref,
              w2_ref, b2_ref, g2_ref, be2_ref, w3r_ref, b3_ref,
              out_ref, acc_ref):
    k = pl.program_id(0)

    @pl.when(k == 0)
    def _init():
        acc_ref[...] = jnp.zeros_like(acc_ref)

    x = parts_ref[0] + parts_ref[1]
    acc_ref[...] += lax.dot_general(
        x, w1_ref[...], (((0,), (0,)), ((), ())),
        preferred_element_type=jnp.float32)

    @pl.when(k == N_KSTEPS - 1)
    def _finish():
        h1 = acc_ref[...] + b1_ref[...]
        m1 = jnp.mean(h1, axis=0, keepdims=True)
        v1 = jnp.mean((h1 - m1) ** 2, axis=0, keepdims=True)
        h1 = g1_ref[...] * (h1 - m1) * lax.rsqrt(v1 + EPS) + be1_ref[...]
        h1 = jnp.maximum(h1, 0.0)
        h2 = jnp.dot(h1, w2_ref[...], preferred_element_type=jnp.float32) + b2_ref[...]
        m2 = jnp.mean(h2, axis=0, keepdims=True)
        v2 = jnp.mean((h2 - m2) ** 2, axis=0, keepdims=True)
        h2 = g2_ref[...] * (h2 - m2) * lax.rsqrt(v2 + EPS) + be2_ref[...]
        h2 = jnp.maximum(h2, 0.0)
        p = jnp.sum(h2 * w3r_ref[...], axis=1, keepdims=True)
        out_ref[...] = p + b3_ref[...]


def _make_mlp():
    full = lambda i: (0, 0)
    return pl.pallas_call(
        _mlp_body,
        grid=(N_KSTEPS,),
        in_specs=[
            pl.BlockSpec((2, KB, B), lambda i: (0, i, 0)),
            pl.BlockSpec((KB, H1), lambda i: (i, 0)),
            pl.BlockSpec((1, H1), full),
            pl.BlockSpec((1, H1), full),
            pl.BlockSpec((1, H1), full),
            pl.BlockSpec((H1, H2), full),
            pl.BlockSpec((1, H2), full),
            pl.BlockSpec((1, H2), full),
            pl.BlockSpec((1, H2), full),
            pl.BlockSpec((1, H2), full),
            pl.BlockSpec((1, 128), full),
        ],
        out_specs=pl.BlockSpec((B, 128), full),
        out_shape=jax.ShapeDtypeStruct((B, 128), jnp.float32),
        scratch_shapes=[pltpu.VMEM((B, H1), jnp.float32)],
    )


def kernel(snp, snp_ids, node_graph_ids, filters, W1, b1, gamma1, beta1,
           W2, b2, gamma2, beta2, W3, b3):
    f32 = jnp.float32

    table = _make_prep()(snp, filters)

    pad = NODES_PAD - N_NODES
    ids_p = jnp.concatenate(
        [snp_ids, jnp.full((pad,), N_SNPS, jnp.int32)]).reshape(
            TOT_CHUNKS, CHUNK)
    genes_p = jnp.concatenate(
        [node_graph_ids, jnp.full((pad,), N_GENES - 1, jnp.int32)]).reshape(
            TOT_CHUNKS, CHUNK)
    parts = _make_sc()(table, ids_p, genes_p)
    parts = parts.reshape(2, N_GENES_PAD, B)

    out = _make_mlp()(
        parts, W1,
        b1.reshape(1, H1), gamma1.reshape(1, H1), beta1.reshape(1, H1),
        W2, b2.reshape(1, H2), gamma2.reshape(1, H2), beta2.reshape(1, H2),
        W3.reshape(1, H2), jnp.broadcast_to(b3.reshape(1, 1), (1, 128)),
    )
    preds = out[:, :1]
    return (preds, filters)
